# Initial kernel scaffold; baseline (speedup 1.0000x reference)
import jax
import jax.numpy as jnp
from jax.experimental import pallas as pl


def kernel(x, train_pos_edge_index, dict_node, W1, b1, W2, b2):
    raise NotImplementedError("write your pallas kernel here")


# speedup vs baseline: 11.0015x; 11.0015x over previous
import functools

import jax
import jax.numpy as jnp
from jax import lax
from jax.experimental import pallas as pl
from jax.experimental.pallas import tpu as pltpu
from jax.experimental.pallas import tpu_sc as plsc

N = 10000
D = 128
E = 320000
NC = 2
NS = 16
NW = NC * NS
CHUNK = 128
ACC_ROWS = 10240
GARBAGE = 10200

E_PAD = 323584
P_PAD = 12288
CPW_EDGE = E_PAD // (NW * CHUNK)
CPW_POOL = P_PAD // (NW * CHUNK)
STRIPE = ACC_ROWS // NS

_MESH = plsc.VectorSubcoreMesh(core_axis_name="c", subcore_axis_name="s")


def _fill(ref, value):
    @pl.loop(0, CHUNK)
    def _(r):
        @pl.loop(0, 8)
        def _(cc):
            ref[r, pl.ds(cc * 16, 16)] = jnp.full((16,), value, jnp.float32)


def _zero_acc(zeros_v, acc_sh, sid):
    @pl.loop(0, STRIPE // CHUNK)
    def _(t):
        pltpu.sync_copy(zeros_v, acc_sh.at[pl.ds(sid * STRIPE + t * CHUNK, CHUNK)])


def _make_sc_agg(cpw):
    @functools.partial(
        pl.kernel,
        out_type=jax.ShapeDtypeStruct((NC, ACC_ROWS, 128), jnp.float32),
        mesh=_MESH,
        scratch_types=[
            pltpu.VMEM((cpw, CHUNK), jnp.int32),
            pltpu.VMEM((cpw, CHUNK), jnp.int32),
            pltpu.VMEM((CHUNK, 128), jnp.float32),
            pltpu.VMEM_SHARED((ACC_ROWS, 128), jnp.float32),
        ],
    )
    def k(vals_hbm, src_hbm, dst_hbm, out_hbm, src_v, dst_v, rows_v, acc_sh):
        core = lax.axis_index("c")
        sid = lax.axis_index("s")
        w = core * NS + sid

        _fill(rows_v, 0.0)
        _zero_acc(rows_v, acc_sh, sid)
        pltpu.sync_copy(src_hbm.at[w], src_v)
        pltpu.sync_copy(dst_hbm.at[w], dst_v)
        plsc.subcore_barrier()

        @pl.loop(0, cpw)
        def _(j):
            pltpu.sync_copy(vals_hbm.at[src_v.at[j]], rows_v)
            pltpu.sync_copy(rows_v, acc_sh.at[dst_v.at[j]], add=True)

        plsc.subcore_barrier()
        pltpu.sync_copy(acc_sh.at[pl.ds(sid * STRIPE, STRIPE)],
                        out_hbm.at[core, pl.ds(sid * STRIPE, STRIPE)])

    return k


def _make_sc_count():
    @functools.partial(
        pl.kernel,
        out_type=jax.ShapeDtypeStruct((NC, 2, ACC_ROWS, 128), jnp.float32),
        mesh=_MESH,
        scratch_types=[
            pltpu.VMEM((CPW_EDGE, CHUNK), jnp.int32),
            pltpu.VMEM((CPW_POOL, CHUNK), jnp.int32),
            pltpu.VMEM((CHUNK, 128), jnp.float32),
            pltpu.VMEM((CHUNK, 128), jnp.float32),
            pltpu.VMEM_SHARED((ACC_ROWS, 128), jnp.float32),
        ],
    )
    def k(eidx_hbm, pidx_hbm, out_hbm, eidx_v, pidx_v, ones_v, zeros_v, acc_sh):
        core = lax.axis_index("c")
        sid = lax.axis_index("s")
        w = core * NS + sid

        _fill(ones_v, 1.0)
        _fill(zeros_v, 0.0)
        pltpu.sync_copy(eidx_hbm.at[w], eidx_v)
        pltpu.sync_copy(pidx_hbm.at[w], pidx_v)

        _zero_acc(zeros_v, acc_sh, sid)
        plsc.subcore_barrier()

        @pl.loop(0, CPW_EDGE)
        def _(j):
            pltpu.sync_copy(ones_v, acc_sh.at[eidx_v.at[j]], add=True)

        plsc.subcore_barrier()
        pltpu.sync_copy(acc_sh.at[pl.ds(sid * STRIPE, STRIPE)],
                        out_hbm.at[core, 0, pl.ds(sid * STRIPE, STRIPE)])

        _zero_acc(zeros_v, acc_sh, sid)
        plsc.subcore_barrier()

        @pl.loop(0, CPW_POOL)
        def _(j):
            pltpu.sync_copy(ones_v, acc_sh.at[pidx_v.at[j]], add=True)

        plsc.subcore_barrier()
        pltpu.sync_copy(acc_sh.at[pl.ds(sid * STRIPE, STRIPE)],
                        out_hbm.at[core, 1, pl.ds(sid * STRIPE, STRIPE)])

    return k


_SC_AGG_EDGE = _make_sc_agg(CPW_EDGE)
_SC_AGG_POOL = _make_sc_agg(CPW_POOL)
_SC_COUNT = _make_sc_count()


def _mm_body(x_ref, w_ref, o_ref):
    o_ref[...] = jnp.dot(x_ref[...], w_ref[...],
                         preferred_element_type=jnp.float32)


def _scale_body(cnt_ref, xw_ref, y_ref, dinv_ref):
    deg = cnt_ref[0, 0, :N, 0:1] + cnt_ref[1, 0, :N, 0:1] + 1.0
    dinv = lax.rsqrt(jnp.maximum(deg, 1.0))
    dinv_ref[...] = dinv
    y_ref[...] = dinv * xw_ref[...]


def _mid_body(p_ref, y_ref, dinv_ref, b_ref, w_ref, o_ref):
    tot = p_ref[0, :N, :] + p_ref[1, :N, :] + y_ref[...]
    h = jnp.maximum(dinv_ref[...] * tot + b_ref[...], 0.0)
    o_ref[...] = dinv_ref[...] * jnp.dot(h, w_ref[...],
                                         preferred_element_type=jnp.float32)


def _final_body(p_ref, y_ref, dinv_ref, b_ref, o_ref):
    tot = p_ref[0, :N, :] + p_ref[1, :N, :] + y_ref[...]
    o_ref[...] = jnp.maximum(dinv_ref[...] * tot + b_ref[...], 0.0)


def _div_body(sp_ref, cp_ref, z_ref):
    s = sp_ref[0, :N, :] + sp_ref[1, :N, :]
    cnt = cp_ref[0, 1, :N, 0:1] + cp_ref[1, 1, :N, 0:1]
    z_ref[...] = s / jnp.maximum(cnt, 1.0)


def _f32(shape):
    return jax.ShapeDtypeStruct(shape, jnp.float32)


def kernel(x, train_pos_edge_index, dict_node, W1, b1, W2, b2):
    src = train_pos_edge_index[0].astype(jnp.int32)
    dst = train_pos_edge_index[1].astype(jnp.int32)

    src_p = jnp.concatenate(
        [src, jnp.zeros((E_PAD - E,), jnp.int32)]).reshape(NW, CPW_EDGE, CHUNK)
    dst_p = jnp.concatenate(
        [dst, jnp.full((E_PAD - E,), GARBAGE, jnp.int32)]).reshape(NW, CPW_EDGE, CHUNK)

    pool_src = jnp.concatenate(
        [jnp.arange(N, dtype=jnp.int32),
         jnp.zeros((P_PAD - N,), jnp.int32)]).reshape(NW, CPW_POOL, CHUNK)
    pool_dst = jnp.concatenate(
        [dict_node.astype(jnp.int32),
         jnp.full((P_PAD - N,), GARBAGE, jnp.int32)]).reshape(NW, CPW_POOL, CHUNK)

    counts = _SC_COUNT(dst_p, pool_dst)
    xw = pl.pallas_call(_mm_body, out_shape=_f32((N, D)))(x, W1)
    y1, dinv = pl.pallas_call(
        _scale_body, out_shape=(_f32((N, D)), _f32((N, 1))))(counts, xw)

    p1 = _SC_AGG_EDGE(y1, src_p, dst_p)
    y2 = pl.pallas_call(_mid_body, out_shape=_f32((N, D)))(
        p1, y1, dinv, b1.reshape(1, D), W2)

    p2 = _SC_AGG_EDGE(y2, src_p, dst_p)
    h2 = pl.pallas_call(_final_body, out_shape=_f32((N, D)))(
        p2, y2, dinv, b2.reshape(1, D))

    sp = _SC_AGG_POOL(h2, pool_src, pool_dst)
    z = pl.pallas_call(_div_body, out_shape=_f32((N, D)))(sp, counts)
    return z
